# baseline (speedup 1.0000x reference)
import dataclasses
import functools

import jax
import jax.numpy as jnp
from jax import lax
from jax.experimental import pallas as pl
from jax.experimental.pallas import tpu as pltpu
from jax.experimental.pallas import tpu_sc as plsc

S = 128
C = 10000
N = 10000
NNZ = 320000
L = 16
NC = 2
NS = 16
NW = NC * NS
CH = 96
NB = 3
CPT = 105
EPT = CH * CPT
NNZP = NW * EPT
EREC = 3 * CH
NP = 10240
ZR = 128
RPT = NP // NS
ZB = NP // S


def _prep_body(ll_ref, out_ref):
    out_ref[...] = jnp.exp(ll_ref[...]).T


def _tc_prep(ll):
    return pl.pallas_call(
        _prep_body,
        out_shape=jax.ShapeDtypeStruct((C, S), jnp.float32),
    )(ll)


def _sc_body(probs_hbm, epack_hbm, acc_hbm, z_hbm,
             e0, e1, e2, rv0, rv1, rv2, g0, g1, g2, z_v,
             acc_sh,
             se0, se1, se2, sg0, sg1, sg2, ss0, ss1, ss2):
    cid = lax.axis_index("c")
    sid = lax.axis_index("s")
    wid = sid * NC + cid

    ebufs = (e0, e1, e2)
    rbufs = (rv0, rv1, rv2)
    gbufs = (g0, g1, g2)
    esems = (se0, se1, se2)
    gsems = (sg0, sg1, sg2)
    ssems = (ss0, ss1, ss2)

    @pl.loop(0, CH)
    def _(r):
        for f in range(S // L):
            g0[r, pl.ds(f * L, L)] = jnp.zeros((L,), jnp.float32)

    @pl.loop(0, ZB)
    def _(r):
        for f in range(S // L):
            z_v[r, pl.ds(f * L, L)] = jnp.zeros((L,), jnp.float32)

    base = sid * RPT

    @pl.loop(0, RPT // CH)
    def _(k):
        pltpu.sync_copy(g0, acc_sh.at[pl.ds(base + k * CH, CH)])

    if RPT % CH:
        pltpu.sync_copy(
            g0.at[pl.ds(0, RPT % CH)],
            acc_sh.at[pl.ds(base + (RPT // CH) * CH, RPT % CH)])

    cbase = wid * CPT

    pltpu.async_copy(epack_hbm.at[pl.ds(cbase * EREC, EREC)], e0, se0)
    pltpu.async_copy(epack_hbm.at[pl.ds((cbase + 1) * EREC, EREC)], e1, se1)
    pltpu.make_async_copy(epack_hbm.at[pl.ds(0, EREC)], e0, se0).wait()
    pltpu.async_copy(probs_hbm.at[e0.at[pl.ds(0, CH)]], g0, sg0)

    plsc.subcore_barrier()

    def chunk_step(j, i):
        i1 = (i + 1) % NB
        i2 = (i + 2) % NB
        x = gbufs[i]
        rv = rbufs[i]
        eb = ebufs[i]

        @pl.when(j >= 2)
        def _():
            pltpu.make_async_copy(
                gbufs[i1], acc_sh.at[rbufs[i1]], ssems[i1]).wait()

        @pl.when(j <= CPT - 3)
        def _():
            pltpu.async_copy(
                epack_hbm.at[pl.ds((cbase + j + 2) * EREC, EREC)],
                ebufs[i2], esems[i2])

        @pl.when(j <= CPT - 2)
        def _():
            pltpu.make_async_copy(
                epack_hbm.at[pl.ds(0, EREC)], ebufs[i1], esems[i1]).wait()
            pltpu.async_copy(
                probs_hbm.at[ebufs[i1].at[pl.ds(0, CH)]], gbufs[i1],
                gsems[i1])

        pltpu.make_async_copy(
            probs_hbm.at[eb.at[pl.ds(0, CH)]], x, gsems[i]).wait()

        @pl.loop(0, CH // L)
        def _(g):
            rg = eb[pl.ds(CH + g * L, L)]
            rv[pl.ds(g * L, L)] = rg
            wg = jnp.exp(plsc.bitcast(eb[pl.ds(2 * CH + g * L, L)],
                                      jnp.float32))
            plsc.addupdate_scatter(
                z_v, [lax.shift_right_logical(rg, 7),
                      lax.bitwise_and(rg, 127)], wg)
            for u in range(L):
                e = g * L + u
                wv = jnp.full((L,), wg[u], dtype=jnp.float32)
                for f in range(S // L):
                    x[e, pl.ds(f * L, L)] = x[e, pl.ds(f * L, L)] * wv

        pltpu.async_copy(x, acc_sh.at[rv], ssems[i], add=True)

    @pl.loop(0, CPT // NB)
    def _(t):
        for i in range(NB):
            chunk_step(t * NB + i, i)

    for i in ((CPT - 2) % NB, (CPT - 1) % NB):
        pltpu.make_async_copy(gbufs[i], acc_sh.at[rbufs[i]],
                              ssems[i]).wait()

    pltpu.sync_copy(z_v, z_hbm.at[cid].at[sid])

    plsc.subcore_barrier()

    @pl.loop(0, RPT // ZR)
    def _(k):
        pltpu.sync_copy(acc_sh.at[pl.ds(base + k * ZR, ZR)],
                        acc_hbm.at[cid].at[pl.ds(base + k * ZR, ZR)])


def _sc_aggregate(probsT, epack):
    mesh = plsc.VectorSubcoreMesh(core_axis_name="c", subcore_axis_name="s")
    cp = pltpu.CompilerParams()
    if "needs_layout_passes" in pltpu.CompilerParams.__dataclass_fields__:
        cp = dataclasses.replace(cp, needs_layout_passes=False)
    kern = pl.kernel(
        _sc_body,
        out_type=(jax.ShapeDtypeStruct((NC, NP, S), jnp.float32),
                  jax.ShapeDtypeStruct((NC, NS, ZB, S), jnp.float32)),
        mesh=mesh,
        compiler_params=cp,
        scratch_types=(
            [pltpu.VMEM((EREC,), jnp.int32) for _ in range(NB)]
            + [pltpu.VMEM((CH,), jnp.int32) for _ in range(NB)]
            + [pltpu.VMEM((CH, S), jnp.float32) for _ in range(NB)]
            + [pltpu.VMEM((ZB, S), jnp.float32),
               pltpu.VMEM_SHARED((NP, S), jnp.float32)]
            + [pltpu.SemaphoreType.DMA for _ in range(3 * NB)]
        ),
    )
    return kern(probsT, epack)


def _fin_body(acc_ref, z_ref, out_ref):
    t = acc_ref[0, :N] + acc_ref[1, :N]
    z = z_ref[...].reshape(NW, ZB * S).sum(axis=0).reshape(NP)[:N]
    out_ref[...] = (jnp.log(t) - jnp.log(z)[:, None]).T


def _tc_finalize(acc, z):
    return pl.pallas_call(
        _fin_body,
        out_shape=jax.ShapeDtypeStruct((S, N), jnp.float32),
    )(acc, z)


def kernel(ll, w_rows, w_cols, w_data):
    pad = NNZP - NNZ
    cols = jnp.concatenate([w_cols.astype(jnp.int32),
                            jnp.zeros((pad,), jnp.int32)]).reshape(-1, CH)
    rows = jnp.concatenate([w_rows.astype(jnp.int32),
                            jnp.zeros((pad,), jnp.int32)]).reshape(-1, CH)
    wbits = lax.bitcast_convert_type(
        jnp.concatenate([w_data, jnp.full((pad,), -1e30, jnp.float32)]),
        jnp.int32).reshape(-1, CH)
    epack = jnp.stack([cols, rows, wbits], axis=1).reshape(-1)
    probsT = _tc_prep(ll)
    acc, z = _sc_aggregate(probsT, epack)
    return _tc_finalize(acc, z)


# speedup vs baseline: 1.6110x; 1.6110x over previous
import dataclasses
import functools

import jax
import jax.numpy as jnp
from jax import lax
from jax.experimental import pallas as pl
from jax.experimental.pallas import tpu as pltpu
from jax.experimental.pallas import tpu_sc as plsc

S = 128
C = 10000
N = 10000
NNZ = 320000
L = 16
NC = 2
NS = 16
NW = NC * NS
CH = 96
NB = 3
CPT = 105
EPT = CH * CPT
NNZP = NW * EPT
EREC = 3 * CH
NP = 10240
ZR = 128
RPT = NP // NS
ZB = NP // S


def _prep_body(ll_ref, out_ref):
    out_ref[...] = jnp.exp(ll_ref[...]).T


def _tc_prep(ll):
    return pl.pallas_call(
        _prep_body,
        out_shape=jax.ShapeDtypeStruct((C, S), jnp.float32),
    )(ll)


def _sc_body(probs_hbm, epack_hbm, acc_hbm, z_hbm,
             e0, e1, e2, rv0, rv1, rv2, g0, g1, g2, z_v,
             acc_sh,
             se0, se1, se2, sg0, sg1, sg2, ss0, ss1, ss2):
    cid = lax.axis_index("c")
    sid = lax.axis_index("s")
    wid = sid * NC + cid

    ebufs = (e0, e1, e2)
    rbufs = (rv0, rv1, rv2)
    gbufs = (g0, g1, g2)
    esems = (se0, se1, se2)
    gsems = (sg0, sg1, sg2)
    ssems = (ss0, ss1, ss2)

    @pl.loop(0, CH)
    def _(r):
        for f in range(S // L):
            g0[r, pl.ds(f * L, L)] = jnp.zeros((L,), jnp.float32)

    @pl.loop(0, ZB)
    def _(r):
        for f in range(S // L):
            z_v[r, pl.ds(f * L, L)] = jnp.zeros((L,), jnp.float32)

    base = sid * RPT

    @pl.loop(0, RPT // CH)
    def _(k):
        pltpu.sync_copy(g0, acc_sh.at[pl.ds(base + k * CH, CH)])

    if RPT % CH:
        pltpu.sync_copy(
            g0.at[pl.ds(0, RPT % CH)],
            acc_sh.at[pl.ds(base + (RPT // CH) * CH, RPT % CH)])

    cbase = wid * CPT

    pltpu.async_copy(epack_hbm.at[pl.ds(cbase * EREC, EREC)], e0, se0)
    pltpu.async_copy(epack_hbm.at[pl.ds((cbase + 1) * EREC, EREC)], e1, se1)
    pltpu.make_async_copy(epack_hbm.at[pl.ds(0, EREC)], e0, se0).wait()
    pltpu.async_copy(probs_hbm.at[e0.at[pl.ds(0, CH)]], g0, sg0)

    plsc.subcore_barrier()

    def chunk_step(j, i):
        i1 = (i + 1) % NB
        i2 = (i + 2) % NB
        x = gbufs[i]
        rv = rbufs[i]
        eb = ebufs[i]

        @pl.when(j >= 2)
        def _():
            pltpu.make_async_copy(
                gbufs[i1], acc_sh.at[rbufs[i1]], ssems[i1]).wait()

        @pl.when(j <= CPT - 3)
        def _():
            pltpu.async_copy(
                epack_hbm.at[pl.ds((cbase + j + 2) * EREC, EREC)],
                ebufs[i2], esems[i2])

        @pl.when(j <= CPT - 2)
        def _():
            pltpu.make_async_copy(
                epack_hbm.at[pl.ds(0, EREC)], ebufs[i1], esems[i1]).wait()
            pltpu.async_copy(
                probs_hbm.at[ebufs[i1].at[pl.ds(0, CH)]], gbufs[i1],
                gsems[i1])

        pltpu.make_async_copy(
            probs_hbm.at[eb.at[pl.ds(0, CH)]], x, gsems[i]).wait()

        @pl.loop(0, CH // L)
        def _(g):
            rg = eb[pl.ds(CH + g * L, L)]
            rv[pl.ds(g * L, L)] = rg
            wg = jnp.exp(plsc.bitcast(eb[pl.ds(2 * CH + g * L, L)],
                                      jnp.float32))
            plsc.addupdate_scatter(
                z_v, [lax.shift_right_logical(rg, 7),
                      lax.bitwise_and(rg, 127)], wg)
            for u in range(L):
                e = g * L + u
                wv = jnp.full((L,), wg[u], dtype=jnp.float32)
                for f in range(S // L):
                    x[e, pl.ds(f * L, L)] = x[e, pl.ds(f * L, L)] * wv

        pltpu.async_copy(x, acc_sh.at[rv], ssems[i], add=True)

    @pl.loop(0, CPT // NB)
    def _(t):
        for i in range(NB):
            chunk_step(t * NB + i, i)

    for i in ((CPT - 2) % NB, (CPT - 1) % NB):
        pltpu.make_async_copy(gbufs[i], acc_sh.at[rbufs[i]],
                              ssems[i]).wait()

    pltpu.sync_copy(z_v, z_hbm.at[cid].at[sid])

    plsc.subcore_barrier()

    @pl.loop(0, RPT // ZR)
    def _(k):
        pltpu.sync_copy(acc_sh.at[pl.ds(base + k * ZR, ZR)],
                        acc_hbm.at[cid].at[pl.ds(base + k * ZR, ZR)])


def _sc_aggregate(probsT, epack):
    mesh = plsc.VectorSubcoreMesh(core_axis_name="c", subcore_axis_name="s")
    cp = pltpu.CompilerParams()
    if "needs_layout_passes" in pltpu.CompilerParams.__dataclass_fields__:
        cp = dataclasses.replace(cp, needs_layout_passes=False)
    kern = pl.kernel(
        _sc_body,
        out_type=(jax.ShapeDtypeStruct((NC, NP, S), jnp.float32),
                  jax.ShapeDtypeStruct((NC, NS, ZB, S), jnp.float32)),
        mesh=mesh,
        compiler_params=cp,
        scratch_types=(
            [pltpu.VMEM((EREC,), jnp.int32) for _ in range(NB)]
            + [pltpu.VMEM((CH,), jnp.int32) for _ in range(NB)]
            + [pltpu.VMEM((CH, S), jnp.float32) for _ in range(NB)]
            + [pltpu.VMEM((ZB, S), jnp.float32),
               pltpu.VMEM_SHARED((NP, S), jnp.float32)]
            + [pltpu.SemaphoreType.DMA for _ in range(3 * NB)]
        ),
    )
    return kern(probsT, epack)


def _fin_body(acc_ref, z_ref, out_ref):
    t = acc_ref[0, :N] + acc_ref[1, :N]
    z = z_ref[...].reshape(NW, ZB * S).sum(axis=0).reshape(NP)[:N]
    out_ref[...] = (jnp.log(t) - jnp.log(z)[:, None]).T


def _tc_finalize(acc, z):
    return pl.pallas_call(
        _fin_body,
        out_shape=jax.ShapeDtypeStruct((S, N), jnp.float32),
    )(acc, z)


def kernel(ll, w_rows, w_cols, w_data):
    pad = NNZP - NNZ
    spread = jnp.arange(pad, dtype=jnp.int32) % N
    cols = jnp.concatenate([w_cols.astype(jnp.int32),
                            spread]).reshape(-1, CH)
    rows = jnp.concatenate([w_rows.astype(jnp.int32),
                            spread]).reshape(-1, CH)
    wbits = lax.bitcast_convert_type(
        jnp.concatenate([w_data, jnp.full((pad,), -1e30, jnp.float32)]),
        jnp.int32).reshape(-1, CH)
    epack = jnp.stack([cols, rows, wbits], axis=1).reshape(-1)
    probsT = _tc_prep(ll)
    acc, z = _sc_aggregate(probsT, epack)
    return _tc_finalize(acc, z)


# speedup vs baseline: 1.6116x; 1.0004x over previous
import dataclasses

import jax
import jax.numpy as jnp
from jax import lax
from jax.experimental import pallas as pl
from jax.experimental.pallas import tpu as pltpu
from jax.experimental.pallas import tpu_sc as plsc

S = 128
C = 10000
N = 10000
NNZ = 320000
L = 16
NC = 2
NS = 16
NW = NC * NS
CH = 96
NB = 3
CPT = 105
EPT = CH * CPT
NNZP = NW * EPT
EREC = 3 * CH
NP = 10240
ZR = 128
RPT = NP // NS
ZB = NP // S


def _prep_body(ll_ref, out_ref):
    out_ref[...] = jnp.exp(ll_ref[...]).T


def _tc_prep(ll):
    return pl.pallas_call(
        _prep_body,
        out_shape=jax.ShapeDtypeStruct((C, S), jnp.float32),
    )(ll)


def _sc_body(probs_hbm, epack_hbm, acc_hbm, z_hbm,
             e0, e1, e2, rv0, rv1, rv2, g0, g1, g2, z_v,
             acc_sh,
             se0, se1, se2, sg0, sg1, sg2, ss0, ss1, ss2):
    cid = lax.axis_index("c")
    sid = lax.axis_index("s")
    wid = sid * NC + cid

    ebufs = (e0, e1, e2)
    rbufs = (rv0, rv1, rv2)
    gbufs = (g0, g1, g2)
    esems = (se0, se1, se2)
    gsems = (sg0, sg1, sg2)
    ssems = (ss0, ss1, ss2)

    @pl.loop(0, CH)
    def _(r):
        for f in range(S // L):
            g0[r, pl.ds(f * L, L)] = jnp.zeros((L,), jnp.float32)

    @pl.loop(0, ZB)
    def _(r):
        for f in range(S // L):
            z_v[r, pl.ds(f * L, L)] = jnp.zeros((L,), jnp.float32)

    base = sid * RPT

    @pl.loop(0, RPT // CH)
    def _(k):
        pltpu.sync_copy(g0, acc_sh.at[pl.ds(base + k * CH, CH)])

    if RPT % CH:
        pltpu.sync_copy(
            g0.at[pl.ds(0, RPT % CH)],
            acc_sh.at[pl.ds(base + (RPT // CH) * CH, RPT % CH)])

    cbase = wid * CPT

    pltpu.async_copy(epack_hbm.at[pl.ds(cbase * EREC, EREC)], e0, se0)
    pltpu.async_copy(epack_hbm.at[pl.ds((cbase + 1) * EREC, EREC)], e1, se1)
    pltpu.make_async_copy(epack_hbm.at[pl.ds(0, EREC)], e0, se0).wait()
    pltpu.async_copy(probs_hbm.at[e0.at[pl.ds(0, CH)]], g0, sg0)

    plsc.subcore_barrier()

    def chunk_step(j, i):
        i1 = (i + 1) % NB
        i2 = (i + 2) % NB
        x = gbufs[i]
        rv = rbufs[i]
        eb = ebufs[i]

        @pl.when(j >= 2)
        def _():
            pltpu.make_async_copy(
                gbufs[i1], acc_sh.at[rbufs[i1]], ssems[i1]).wait()

        @pl.when(j <= CPT - 3)
        def _():
            pltpu.async_copy(
                epack_hbm.at[pl.ds((cbase + j + 2) * EREC, EREC)],
                ebufs[i2], esems[i2])

        @pl.when(j <= CPT - 2)
        def _():
            pltpu.make_async_copy(
                epack_hbm.at[pl.ds(0, EREC)], ebufs[i1], esems[i1]).wait()
            pltpu.async_copy(
                probs_hbm.at[ebufs[i1].at[pl.ds(0, CH)]], gbufs[i1],
                gsems[i1])

        pltpu.make_async_copy(
            probs_hbm.at[eb.at[pl.ds(0, CH)]], x, gsems[i]).wait()

        @pl.loop(0, CH // L)
        def _(g):
            rg = eb[pl.ds(CH + g * L, L)]
            rv[pl.ds(g * L, L)] = rg
            wg = jnp.exp(plsc.bitcast(eb[pl.ds(2 * CH + g * L, L)],
                                      jnp.float32))
            plsc.addupdate_scatter(
                z_v, [lax.shift_right_logical(rg, 7),
                      lax.bitwise_and(rg, 127)], wg)
            for u in range(L):
                e = g * L + u
                wv = jnp.full((L,), wg[u], dtype=jnp.float32)
                for f in range(S // L):
                    x[e, pl.ds(f * L, L)] = x[e, pl.ds(f * L, L)] * wv

        pltpu.async_copy(x, acc_sh.at[rv], ssems[i], add=True)

    @pl.loop(0, CPT // NB)
    def _(t):
        for i in range(NB):
            chunk_step(t * NB + i, i)

    for i in ((CPT - 2) % NB, (CPT - 1) % NB):
        pltpu.make_async_copy(gbufs[i], acc_sh.at[rbufs[i]],
                              ssems[i]).wait()

    pltpu.sync_copy(z_v, z_hbm.at[cid].at[sid])

    plsc.subcore_barrier()

    @pl.loop(0, RPT // ZR)
    def _(k):
        pltpu.sync_copy(acc_sh.at[pl.ds(base + k * ZR, ZR)],
                        acc_hbm.at[cid].at[pl.ds(base + k * ZR, ZR)])


def _sc_aggregate(probsT, epack):
    mesh = plsc.VectorSubcoreMesh(core_axis_name="c", subcore_axis_name="s")
    cp = pltpu.CompilerParams()
    if "needs_layout_passes" in pltpu.CompilerParams.__dataclass_fields__:
        cp = dataclasses.replace(cp, needs_layout_passes=False)
    kern = pl.kernel(
        _sc_body,
        out_type=(jax.ShapeDtypeStruct((NC, NP, S), jnp.float32),
                  jax.ShapeDtypeStruct((NC, NS, ZB, S), jnp.float32)),
        mesh=mesh,
        compiler_params=cp,
        scratch_types=(
            [pltpu.VMEM((EREC,), jnp.int32) for _ in range(NB)]
            + [pltpu.VMEM((CH,), jnp.int32) for _ in range(NB)]
            + [pltpu.VMEM((CH, S), jnp.float32) for _ in range(NB)]
            + [pltpu.VMEM((ZB, S), jnp.float32),
               pltpu.VMEM_SHARED((NP, S), jnp.float32)]
            + [pltpu.SemaphoreType.DMA for _ in range(3 * NB)]
        ),
    )
    return kern(probsT, epack)


def _fin_body(acc_ref, z_ref, out_ref):
    t = acc_ref[0, :N] + acc_ref[1, :N]
    z = z_ref[...].reshape(NW, ZB * S).sum(axis=0).reshape(NP)[:N]
    out_ref[...] = (jnp.log(t) - jnp.log(z)[:, None]).T


def _tc_finalize(acc, z):
    return pl.pallas_call(
        _fin_body,
        out_shape=jax.ShapeDtypeStruct((S, N), jnp.float32),
    )(acc, z)


def kernel(ll, w_rows, w_cols, w_data):
    pad = NNZP - NNZ
    spread = jnp.arange(pad, dtype=jnp.int32) % N
    cols = jnp.concatenate([w_cols.astype(jnp.int32),
                            spread]).reshape(-1, CH)
    rows = jnp.concatenate([w_rows.astype(jnp.int32),
                            spread]).reshape(-1, CH)
    wbits = lax.bitcast_convert_type(
        jnp.concatenate([w_data, jnp.full((pad,), -1e30, jnp.float32)]),
        jnp.int32).reshape(-1, CH)
    epack = jnp.stack([cols, rows, wbits], axis=1).reshape(-1)
    probsT = _tc_prep(ll)
    acc, z = _sc_aggregate(probsT, epack)
    return _tc_finalize(acc, z)
